# baseline (speedup 1.0000x reference)
import jax
import jax.numpy as jnp
from jax.experimental import pallas as pl
from jax.experimental.pallas import tpu as pltpu

OVERLAP_THRESH = 0.5
NEG_POS_RATIO = 3
VAR0 = 0.1
VAR1 = 0.2
ROWS_PER_STEP = 4


def _arm_loss_kernel(loc_ref, conf_ref, priors2_ref, priorsr_ref, truths_ref,
                     ll_ref, lc_ref, np_ref, proxy_s, np_s):
    b = pl.program_id(0)
    n_steps = pl.num_programs(0)
    P = priors2_ref.shape[1]
    T = truths_ref.shape[1]
    S = loc_ref.shape[2]
    L = loc_ref.shape[3]
    R = loc_ref.shape[0]

    @pl.when(b == 0)
    def _init():
        ll_ref[...] = jnp.zeros((1, 1, 1), jnp.float32)
        lc_ref[...] = jnp.zeros((1, 1, 1), jnp.float32)
        np_ref[...] = jnp.zeros((1, 1, 1), jnp.float32)

    pcx = priors2_ref[0:1, :]
    pcy = priors2_ref[1:2, :]
    pw = priors2_ref[2:3, :]
    ph = priors2_ref[3:4, :]
    pxmin = pcx - pw / 2.0
    pymin = pcy - ph / 2.0
    pxmax = pcx + pw / 2.0
    pymax = pcy + ph / 2.0
    area_p = (pxmax - pxmin) * (pymax - pymin)

    rcx = priorsr_ref[0]
    rcy = priorsr_ref[1]
    rw = priorsr_ref[2]
    rh = priorsr_ref[3]

    iota_p = jax.lax.broadcasted_iota(jnp.int32, (1, P), 1)
    iota_tp = jax.lax.broadcasted_iota(jnp.int32, (T, P), 0)

    def sl1(d):
        a = jnp.abs(d)
        return jnp.where(a < 1.0, 0.5 * d * d, a - 0.5)

    ll_acc = jnp.float32(0.0)
    ce_acc = jnp.float32(0.0)
    np_acc = jnp.float32(0.0)

    for i in range(R):
        truths = truths_ref[i]
        txmin = truths[:, 0:1]
        tymin = truths[:, 1:2]
        txmax = truths[:, 2:3]
        tymax = truths[:, 3:4]
        area_t = (txmax - txmin) * (tymax - tymin)

        iw = jnp.clip(jnp.minimum(txmax, pxmax) - jnp.maximum(txmin, pxmin),
                      0.0, None)
        ih = jnp.clip(jnp.minimum(tymax, pymax) - jnp.maximum(tymin, pymin),
                      0.0, None)
        inter = iw * ih
        ov = inter / (area_t + area_p - inter)

        bto = jnp.max(ov, axis=0, keepdims=True)
        bti = jnp.argmax(ov, axis=0).reshape(1, P)
        bp = jnp.argmax(ov, axis=1).reshape(T, 1)

        forced_t = jnp.max(jnp.where(bp == iota_p, iota_tp, -1), axis=0,
                           keepdims=True)
        forced_any = forced_t >= 0
        bto = jnp.where(forced_any, 2.0, bto)
        bti = jnp.where(forced_any, forced_t, bti)

        m = (bti == iota_tp).astype(jnp.float32)
        matched = jax.lax.dot_general(
            truths, m, (((0,), (0,)), ((), ())),
            preferred_element_type=jnp.float32)

        btor = bto.reshape(S, L)
        pos = btor >= OVERLAP_THRESH
        posf = pos.astype(jnp.float32)

        mx0 = matched[0:1, :].reshape(S, L)
        my0 = matched[1:2, :].reshape(S, L)
        mx1 = matched[2:3, :].reshape(S, L)
        my1 = matched[3:4, :].reshape(S, L)

        g_cx = ((mx0 + mx1) / 2.0 - rcx) / (VAR0 * rw)
        g_cy = ((my0 + my1) / 2.0 - rcy) / (VAR0 * rh)
        g_w = jnp.log((mx1 - mx0) / rw) / VAR1
        g_h = jnp.log((my1 - my0) / rh) / VAR1

        loss_l = jnp.sum(
            (sl1(loc_ref[i, 0] - g_cx) + sl1(loc_ref[i, 1] - g_cy)
             + sl1(loc_ref[i, 2] - g_w) + sl1(loc_ref[i, 3] - g_h)) * posf)

        x0 = conf_ref[i, 0]
        x1 = conf_ref[i, 1]
        mx = jnp.maximum(x0, x1)
        lse = mx + jnp.log(jnp.exp(x0 - mx) + jnp.exp(x1 - mx))
        proxy = jnp.where(pos, 0.0, lse - x0)

        num_pos = jnp.sum(posf)
        ce_pos = jnp.sum(jnp.where(pos, lse - x1, 0.0))

        proxy_s[b * R + i] = proxy
        np_s[b * R + i] = jnp.full((S, 128), num_pos, jnp.float32)

        ll_acc += loss_l
        ce_acc += ce_pos
        np_acc += num_pos

    ll_ref[...] += ll_acc.reshape(1, 1, 1)
    lc_ref[...] += ce_acc.reshape(1, 1, 1)
    np_ref[...] += np_acc.reshape(1, 1, 1)

    @pl.when(b == n_steps - 1)
    def _select():
        proxy_all = proxy_s[...]
        bits = jax.lax.bitcast_convert_type(proxy_all, jnp.int32)
        np_i = np_s[:, 0:1, 0:1].astype(jnp.int32)
        k = jnp.minimum(NEG_POS_RATIO * np_i, P - np_i)

        def vstep(_, carry):
            lo, hi = carry
            mid = lo + (hi - lo) // 2
            cnt = jnp.sum((bits > mid).astype(jnp.int32), axis=(1, 2),
                          keepdims=True)
            take_hi = cnt < k
            return (jnp.where(take_hi, lo, mid + 1),
                    jnp.where(take_hi, mid, hi))

        nb = proxy_s.shape[0]
        lo0 = jnp.zeros((nb, 1, 1), jnp.int32)
        hi0 = jnp.full((nb, 1, 1), 0x7F7FFFFF, jnp.int32)
        lo, hi = jax.lax.fori_loop(0, 32, vstep, (lo0, hi0))
        vk = hi

        gt = bits > vk
        count_gt = jnp.sum(gt.astype(jnp.int32), axis=(1, 2), keepdims=True)
        needed = (k - count_gt).astype(jnp.float32)
        vkf = jax.lax.bitcast_convert_type(vk, jnp.float32)

        ce_neg = (jnp.sum(jnp.where(gt, proxy_all, 0.0))
                  + jnp.sum(needed * vkf))
        lc_ref[...] += ce_neg.reshape(1, 1, 1)


@jax.jit
def kernel(loc_pred, conf_pred, priors, targets):
    B, P, _ = loc_pred.shape
    T = targets.shape[1]
    S = 8
    L = P // S
    R = ROWS_PER_STEP
    locT = jnp.transpose(loc_pred, (0, 2, 1)).reshape(B, 4, S, L)
    confT = jnp.transpose(conf_pred, (0, 2, 1)).reshape(B, 2, S, L)
    priorsT = jnp.transpose(priors, (1, 0))
    priorsR = priorsT.reshape(4, S, L)
    truths = targets[:, :, :4]

    out_shape = [jax.ShapeDtypeStruct((1, 1, 1), jnp.float32)] * 3
    scalar_spec = pl.BlockSpec((1, 1, 1), lambda b: (0, 0, 0))
    ll, lc, npos = pl.pallas_call(
        _arm_loss_kernel,
        grid=(B // R,),
        in_specs=[
            pl.BlockSpec((R, 4, S, L), lambda b: (b, 0, 0, 0)),
            pl.BlockSpec((R, 2, S, L), lambda b: (b, 0, 0, 0)),
            pl.BlockSpec((4, P), lambda b: (0, 0)),
            pl.BlockSpec((4, S, L), lambda b: (0, 0, 0)),
            pl.BlockSpec((R, T, 4), lambda b: (b, 0, 0)),
        ],
        out_specs=[scalar_spec, scalar_spec, scalar_spec],
        out_shape=out_shape,
        scratch_shapes=[
            pltpu.VMEM((B, S, L), jnp.float32),
            pltpu.VMEM((B, S, 128), jnp.float32),
        ],
        compiler_params=pltpu.CompilerParams(
            dimension_semantics=("arbitrary",),
            allow_input_fusion=[True, True, True, True, True]),
    )(locT, confT, priorsT, priorsR, truths)

    total = npos[0, 0, 0]
    return (ll[0, 0, 0] / total, lc[0, 0, 0] / total)


# speedup vs baseline: 1.2666x; 1.2666x over previous
import jax
import jax.numpy as jnp
from jax.experimental import pallas as pl
from jax.experimental.pallas import tpu as pltpu

OVERLAP_THRESH = 0.5
NEG_POS_RATIO = 3
VAR0 = 0.1
VAR1 = 0.2
ROWS_PER_STEP = 2


def _arm_loss_kernel(loc_ref, conf_ref, priors2_ref, priorsr_ref, truths_ref,
                     ll_ref, lc_ref, np_ref, proxy_s, np_s):
    b = pl.program_id(0)
    n_steps = pl.num_programs(0)
    P = priors2_ref.shape[1]
    T = truths_ref.shape[1]
    S = loc_ref.shape[2]
    L = loc_ref.shape[3]
    R = loc_ref.shape[0]

    @pl.when(b == 0)
    def _init():
        ll_ref[...] = jnp.zeros((1, 1, 1), jnp.float32)
        lc_ref[...] = jnp.zeros((1, 1, 1), jnp.float32)
        np_ref[...] = jnp.zeros((1, 1, 1), jnp.float32)

    pcx = priors2_ref[0:1, :]
    pcy = priors2_ref[1:2, :]
    pw = priors2_ref[2:3, :]
    ph = priors2_ref[3:4, :]
    pxmin = pcx - pw / 2.0
    pymin = pcy - ph / 2.0
    pxmax = pcx + pw / 2.0
    pymax = pcy + ph / 2.0
    area_p = (pxmax - pxmin) * (pymax - pymin)

    rcx = priorsr_ref[0]
    rcy = priorsr_ref[1]
    rw = priorsr_ref[2]
    rh = priorsr_ref[3]

    iota_p = jax.lax.broadcasted_iota(jnp.int32, (1, P), 1)
    iota_tp = jax.lax.broadcasted_iota(jnp.int32, (T, P), 0)

    def sl1(d):
        a = jnp.abs(d)
        return jnp.where(a < 1.0, 0.5 * d * d, a - 0.5)

    ll_acc = jnp.float32(0.0)
    ce_acc = jnp.float32(0.0)
    np_acc = jnp.float32(0.0)

    for i in range(R):
        truths = truths_ref[i]
        txmin = truths[:, 0:1]
        tymin = truths[:, 1:2]
        txmax = truths[:, 2:3]
        tymax = truths[:, 3:4]
        area_t = (txmax - txmin) * (tymax - tymin)

        iw = jnp.clip(jnp.minimum(txmax, pxmax) - jnp.maximum(txmin, pxmin),
                      0.0, None)
        ih = jnp.clip(jnp.minimum(tymax, pymax) - jnp.maximum(tymin, pymin),
                      0.0, None)
        inter = iw * ih
        ov = inter / (area_t + area_p - inter)

        bto = jnp.max(ov, axis=0, keepdims=True)
        bti = jnp.argmax(ov, axis=0).reshape(1, P)
        bp = jnp.argmax(ov, axis=1).reshape(T, 1)

        forced_t = jnp.max(jnp.where(bp == iota_p, iota_tp, -1), axis=0,
                           keepdims=True)
        forced_any = forced_t >= 0
        bto = jnp.where(forced_any, 2.0, bto)
        bti = jnp.where(forced_any, forced_t, bti)

        m = (bti == iota_tp).astype(jnp.float32)
        matched = jax.lax.dot_general(
            truths, m, (((0,), (0,)), ((), ())),
            preferred_element_type=jnp.float32)

        btor = bto.reshape(S, L)
        pos = btor >= OVERLAP_THRESH
        posf = pos.astype(jnp.float32)

        mx0 = matched[0:1, :].reshape(S, L)
        my0 = matched[1:2, :].reshape(S, L)
        mx1 = matched[2:3, :].reshape(S, L)
        my1 = matched[3:4, :].reshape(S, L)

        g_cx = ((mx0 + mx1) / 2.0 - rcx) / (VAR0 * rw)
        g_cy = ((my0 + my1) / 2.0 - rcy) / (VAR0 * rh)
        g_w = jnp.log((mx1 - mx0) / rw) / VAR1
        g_h = jnp.log((my1 - my0) / rh) / VAR1

        loss_l = jnp.sum(
            (sl1(loc_ref[i, 0] - g_cx) + sl1(loc_ref[i, 1] - g_cy)
             + sl1(loc_ref[i, 2] - g_w) + sl1(loc_ref[i, 3] - g_h)) * posf)

        x0 = conf_ref[i, 0]
        x1 = conf_ref[i, 1]
        mx = jnp.maximum(x0, x1)
        lse = mx + jnp.log(jnp.exp(x0 - mx) + jnp.exp(x1 - mx))
        proxy = jnp.where(pos, 0.0, lse - x0)

        num_pos = jnp.sum(posf)
        ce_pos = jnp.sum(jnp.where(pos, lse - x1, 0.0))

        proxy_s[b * R + i] = proxy
        np_s[b * R + i] = jnp.full((S, 128), num_pos, jnp.float32)

        ll_acc += loss_l
        ce_acc += ce_pos
        np_acc += num_pos

    ll_ref[...] += ll_acc.reshape(1, 1, 1)
    lc_ref[...] += ce_acc.reshape(1, 1, 1)
    np_ref[...] += np_acc.reshape(1, 1, 1)

    @pl.when(b == n_steps - 1)
    def _select():
        proxy_all = proxy_s[...]
        bits = jax.lax.bitcast_convert_type(proxy_all, jnp.int32)
        np_i = np_s[:, 0:1, 0:1].astype(jnp.int32)
        k = jnp.minimum(NEG_POS_RATIO * np_i, P - np_i)

        def vstep(_, carry):
            lo, hi = carry
            mid = lo + (hi - lo) // 2
            cnt = jnp.sum((bits > mid).astype(jnp.int32), axis=(1, 2),
                          keepdims=True)
            take_hi = cnt < k
            return (jnp.where(take_hi, lo, mid + 1),
                    jnp.where(take_hi, mid, hi))

        nb = proxy_s.shape[0]
        lo0 = jnp.zeros((nb, 1, 1), jnp.int32)
        hi0 = jnp.full((nb, 1, 1), 0x7F7FFFFF, jnp.int32)
        lo, hi = jax.lax.fori_loop(0, 32, vstep, (lo0, hi0))
        vk = hi

        gt = bits > vk
        count_gt = jnp.sum(gt.astype(jnp.int32), axis=(1, 2), keepdims=True)
        needed = (k - count_gt).astype(jnp.float32)
        vkf = jax.lax.bitcast_convert_type(vk, jnp.float32)

        ce_neg = (jnp.sum(jnp.where(gt, proxy_all, 0.0))
                  + jnp.sum(needed * vkf))
        lc_ref[...] += ce_neg.reshape(1, 1, 1)


@jax.jit
def kernel(loc_pred, conf_pred, priors, targets):
    B, P, _ = loc_pred.shape
    T = targets.shape[1]
    S = 8
    L = P // S
    R = ROWS_PER_STEP
    locT = jnp.transpose(loc_pred, (0, 2, 1)).reshape(B, 4, S, L)
    confT = jnp.transpose(conf_pred, (0, 2, 1)).reshape(B, 2, S, L)
    priorsT = jnp.transpose(priors, (1, 0))
    priorsR = priorsT.reshape(4, S, L)
    truths = targets[:, :, :4]

    out_shape = [jax.ShapeDtypeStruct((1, 1, 1), jnp.float32)] * 3
    scalar_spec = pl.BlockSpec((1, 1, 1), lambda b: (0, 0, 0))
    ll, lc, npos = pl.pallas_call(
        _arm_loss_kernel,
        grid=(B // R,),
        in_specs=[
            pl.BlockSpec((R, 4, S, L), lambda b: (b, 0, 0, 0)),
            pl.BlockSpec((R, 2, S, L), lambda b: (b, 0, 0, 0)),
            pl.BlockSpec((4, P), lambda b: (0, 0)),
            pl.BlockSpec((4, S, L), lambda b: (0, 0, 0)),
            pl.BlockSpec((R, T, 4), lambda b: (b, 0, 0)),
        ],
        out_specs=[scalar_spec, scalar_spec, scalar_spec],
        out_shape=out_shape,
        scratch_shapes=[
            pltpu.VMEM((B, S, L), jnp.float32),
            pltpu.VMEM((B, S, 128), jnp.float32),
        ],
        compiler_params=pltpu.CompilerParams(
            dimension_semantics=("arbitrary",),
            allow_input_fusion=[True, True, True, True, True]),
    )(locT, confT, priorsT, priorsR, truths)

    total = npos[0, 0, 0]
    return (ll[0, 0, 0] / total, lc[0, 0, 0] / total)
